# baseline (speedup 1.0000x reference)
import functools
import jax
import jax.numpy as jnp
from jax import lax
from jax.experimental import pallas as pl
from jax.experimental.pallas import tpu as pltpu
from jax.experimental.pallas import tpu_sc as plsc

NUM_BUCKETS = 1000000
DIM = 64
MODEL_DIM = 1024
BATCH = 4
SEQ = 8192

NC, NS, L = 2, 16, 16
NW = NC * NS
TOTAL = BATCH * SEQ
CHUNK = TOTAL // NW
IDX_ROWS = CHUNK // 128
IDX_COLS = 128


def _sc_gather_body(ids_hbm, table_hbm, emb_hbm, ext_v, idx_v, rows_v, sem):
    wid = lax.axis_index("s") * NC + lax.axis_index("c")
    base = wid * CHUNK

    pltpu.sync_copy(ids_hbm.at[pl.ds(base, CHUNK)], ext_v.at[pl.ds(L, CHUNK)])
    at_row_start = (base % SEQ) == 0

    @pl.when(at_row_start)
    def _():
        ext_v[pl.ds(0, L)] = jnp.zeros((L,), jnp.int32)

    @pl.when(jnp.logical_not(at_row_start))
    def _():
        pltpu.sync_copy(ids_hbm.at[pl.ds(base - L, L)], ext_v.at[pl.ds(0, L)])

    for j in range(IDX_ROWS):
        for t in range(IDX_COLS // L):
            i = j * (IDX_COLS // L) + t
            cur = ext_v[pl.ds(L + i * L, L)]
            prev = ext_v[pl.ds(L - 1 + i * L, L)]
            idx_v[j, pl.ds(t * L, L)] = (prev * 31 + cur) % NUM_BUCKETS

    copies = [
        pltpu.async_copy(
            table_hbm.at[idx_v.at[j]],
            rows_v.at[pl.ds(j * IDX_COLS, IDX_COLS)],
            sem,
        )
        for j in range(IDX_ROWS)
    ]
    for c in copies:
        c.wait()
    pltpu.sync_copy(rows_v, emb_hbm.at[pl.ds(base, CHUNK)])


@jax.jit
def _sc_gather(ids_flat, table):
    mesh = plsc.VectorSubcoreMesh(
        core_axis_name="c", subcore_axis_name="s", num_cores=NC, num_subcores=NS
    )
    return pl.kernel(
        _sc_gather_body,
        out_type=jax.ShapeDtypeStruct((TOTAL, DIM), jnp.float32),
        mesh=mesh,
        scratch_types=[
            pltpu.VMEM((CHUNK + L,), jnp.int32),
            pltpu.VMEM((IDX_ROWS, IDX_COLS), jnp.int32),
            pltpu.VMEM((CHUNK, DIM), jnp.float32),
            pltpu.SemaphoreType.DMA,
        ],
        compiler_params=pltpu.CompilerParams(use_tc_tiling_on_sc=False),
    )(ids_flat, table)


ROWS_BLK = 2048


def _proj_body(emb_ref, w_ref, out_ref):
    out_ref[...] = lax.dot_general(
        emb_ref[...].astype(jnp.bfloat16),
        w_ref[...],
        (((1,), (1,)), ((), ())),
        preferred_element_type=jnp.float32,
    )


@jax.jit
def _proj(emb, wb):
    return pl.pallas_call(
        _proj_body,
        grid=(TOTAL // ROWS_BLK,),
        in_specs=[
            pl.BlockSpec((ROWS_BLK, DIM), lambda i: (i, 0)),
            pl.BlockSpec((MODEL_DIM, DIM), lambda i: (0, 0)),
        ],
        out_specs=pl.BlockSpec((ROWS_BLK, MODEL_DIM), lambda i: (i, 0)),
        out_shape=jax.ShapeDtypeStruct((TOTAL, MODEL_DIM), jnp.float32),
    )(emb, wb)


def kernel(input_ids, embedding_weight, proj_weight):
    ids_flat = input_ids.reshape(-1)
    wb = proj_weight.astype(jnp.bfloat16)
    emb = _sc_gather(ids_flat, embedding_weight)
    out = _proj(emb, wb)
    return out.reshape(BATCH, SEQ, MODEL_DIM)


# speedup vs baseline: 1.1127x; 1.1127x over previous
import functools
import jax
import jax.numpy as jnp
from jax import lax
from jax.experimental import pallas as pl
from jax.experimental.pallas import tpu as pltpu
from jax.experimental.pallas import tpu_sc as plsc

NUM_BUCKETS = 1000000
DIM = 64
MODEL_DIM = 1024
BATCH = 4
SEQ = 8192

NC, NS, L = 2, 16, 16
NW = NC * NS
TOTAL = BATCH * SEQ
CHUNK = TOTAL // NW
HALF = CHUNK // 2
IDX_ROWS = CHUNK // 128
IDX_COLS = 128


def _sc_gather_body(ids_hbm, table_hbm, emb_hbm, ext_v, idx_v, rows_v, sem):
    wid = lax.axis_index("s") * NC + lax.axis_index("c")
    base = wid * CHUNK

    pltpu.sync_copy(ids_hbm.at[pl.ds(base, CHUNK)], ext_v.at[pl.ds(L, CHUNK)])
    at_row_start = (base % SEQ) == 0

    @pl.when(at_row_start)
    def _():
        ext_v[pl.ds(0, L)] = jnp.zeros((L,), jnp.int32)

    @pl.when(jnp.logical_not(at_row_start))
    def _():
        pltpu.sync_copy(ids_hbm.at[pl.ds(base - L, L)], ext_v.at[pl.ds(0, L)])

    for j in range(IDX_ROWS):
        for t in range(IDX_COLS // L):
            i = j * (IDX_COLS // L) + t
            cur = ext_v[pl.ds(L + i * L, L)]
            prev = ext_v[pl.ds(L - 1 + i * L, L)]
            idx_v[j, pl.ds(t * L, L)] = (prev * 31 + cur) % NUM_BUCKETS

    for half in range(2):
        copies = [
            pltpu.async_copy(
                table_hbm.at[idx_v.at[half * (IDX_ROWS // 2) + j]],
                rows_v.at[pl.ds(j * IDX_COLS, IDX_COLS)],
                sem,
            )
            for j in range(IDX_ROWS // 2)
        ]
        for c in copies:
            c.wait()
        pltpu.sync_copy(rows_v, emb_hbm.at[pl.ds(base + half * HALF, HALF)])


@jax.jit
def _sc_gather(ids_flat, table_pad):
    mesh = plsc.VectorSubcoreMesh(
        core_axis_name="c", subcore_axis_name="s", num_cores=NC, num_subcores=NS
    )
    return pl.kernel(
        _sc_gather_body,
        out_type=jax.ShapeDtypeStruct((TOTAL, 2 * DIM), jnp.float32),
        mesh=mesh,
        scratch_types=[
            pltpu.VMEM((CHUNK + L,), jnp.int32),
            pltpu.VMEM((IDX_ROWS, IDX_COLS), jnp.int32),
            pltpu.VMEM((HALF, 2 * DIM), jnp.float32),
            pltpu.SemaphoreType.DMA,
        ],
    )(ids_flat, table_pad)


ROWS_BLK = 2048


def _proj_body(emb_ref, w_ref, out_ref):
    out_ref[...] = lax.dot_general(
        emb_ref[:, :DIM],
        w_ref[...],
        (((1,), (1,)), ((), ())),
        preferred_element_type=jnp.float32,
    )


@jax.jit
def _proj(emb, w):
    return pl.pallas_call(
        _proj_body,
        grid=(TOTAL // ROWS_BLK,),
        in_specs=[
            pl.BlockSpec((ROWS_BLK, 2 * DIM), lambda i: (i, 0)),
            pl.BlockSpec((MODEL_DIM, DIM), lambda i: (0, 0)),
        ],
        out_specs=pl.BlockSpec((ROWS_BLK, MODEL_DIM), lambda i: (i, 0)),
        out_shape=jax.ShapeDtypeStruct((TOTAL, MODEL_DIM), jnp.float32),
    )(emb, w)


def kernel(input_ids, embedding_weight, proj_weight):
    ids_flat = input_ids.reshape(-1)
    table_pad = jnp.pad(embedding_weight, ((0, 0), (0, 2 * DIM - DIM)))
    emb = _sc_gather(ids_flat, table_pad)
    out = _proj(emb, proj_weight)
    return out.reshape(BATCH, SEQ, MODEL_DIM)
